# Initial kernel scaffold; baseline (speedup 1.0000x reference)
import jax
import jax.numpy as jnp
from jax.experimental import pallas as pl


def kernel(x, Wq, Wk, Wv, Wo, Wquer, bquer, Wf, Wdw, alpha, beta):
    raise NotImplementedError("write your pallas kernel here")


# speedup vs baseline: 34.4113x; 34.4113x over previous
import functools

import jax
import jax.numpy as jnp
import numpy as np
from jax.experimental import pallas as pl

HEADS = 4
C = 192
INNER = 3 * C
DH = INNER // HEADS
PH = PW = 8
NPP = PH * PW
NH = NW = 28
NPATCH = NH * NW
HW = 224 * 224
PYR_CELLS = 21

PATCHES_PER_STEP = 16
TOK_PER_STEP = PATCHES_PER_STEP * NPP
GRID1 = NPATCH // PATCHES_PER_STEP

PIX_PER_STEP = 3584
GRID3 = HW // PIX_PER_STEP

_F32 = jnp.float32


def _attn_kernel(t_ref, wq_ref, wk_ref, wv_ref, wo_ref, out_ref, sums_ref):
    t = t_ref[...]
    q = jnp.dot(t, wq_ref[...], preferred_element_type=_F32)
    k = jnp.dot(t, wk_ref[...], preferred_element_type=_F32)
    v = jnp.dot(t, wv_ref[...], preferred_element_type=_F32)
    outs = []
    for h in range(HEADS):
        sl = slice(h * DH, (h + 1) * DH)
        qh = q[:, sl].reshape(PATCHES_PER_STEP, NPP, DH)
        kh = k[:, sl].reshape(PATCHES_PER_STEP, NPP, DH)
        vh = v[:, sl].reshape(PATCHES_PER_STEP, NPP, DH)
        s = jax.lax.dot_general(qh, kh, (((2,), (2,)), ((0,), (0,))),
                                preferred_element_type=_F32)
        m = jnp.max(s, axis=-1, keepdims=True)
        p = jnp.exp(s - m)
        z = jnp.sum(p, axis=-1, keepdims=True)
        d = p / z
        oh = jax.lax.dot_general(d, vh, (((2,), (1,)), ((0,), (0,))),
                                 preferred_element_type=_F32)
        outs.append(oh.reshape(TOK_PER_STEP, DH))
    o = jnp.concatenate(outs, axis=1)
    out_ref[...] = jnp.dot(o, wo_ref[...], preferred_element_type=_F32)
    sums_ref[...] = jnp.sum(t.reshape(PATCHES_PER_STEP, NPP, C), axis=1)


def _ctx_kernel(sums_ref, m_ref, wf_ref, fc_ref):
    pooled = jax.lax.dot_general(sums_ref[...], m_ref[...],
                                 (((0,), (0,)), ((), ())),
                                 preferred_element_type=_F32)
    feats = jax.lax.dot_general(wf_ref[...], pooled,
                                (((0,), (0,)), ((), ())),
                                preferred_element_type=_F32)
    mx = jnp.max(feats, axis=-1, keepdims=True)
    e = jnp.exp(feats - mx)
    keys = e / jnp.sum(e, axis=-1, keepdims=True)
    fc_ref[...] = jax.lax.dot_general(feats, keys, (((1,), (1,)), ((), ())),
                                      preferred_element_type=_F32)


def _fmam_kernel(x_ref, spa_ref, fc_ref, wq_ref, bq_ref, wdw_ref, out_ref):
    x = x_ref[...]
    qf = jax.lax.dot_general(wq_ref[...], x, (((0,), (0,)), ((), ())),
                             preferred_element_type=_F32)
    qf = qf + bq_ref[...]
    mx = jnp.max(qf, axis=0, keepdims=True)
    e = jnp.exp(qf - mx)
    qf = e / jnp.sum(e, axis=0, keepdims=True)
    fa = jax.lax.dot_general(fc_ref[...], qf, (((0,), (0,)), ((), ())),
                             preferred_element_type=_F32)
    w0 = wdw_ref[:, 0:1]
    w1 = wdw_ref[:, 1:2]
    out_ref[...] = spa_ref[...] * w0 + fa * w1


def _pool_matrix():
    m = np.zeros((NPATCH, PYR_CELLS), dtype=np.float32)
    col = 0
    for lvl in range(3):
        s = 2 ** lvl
        pps = NH // s
        npx = (224 // s) * (224 // s)
        for i in range(s):
            for j in range(s):
                for ph in range(i * pps, (i + 1) * pps):
                    for pw_ in range(j * pps, (j + 1) * pps):
                        m[ph * NW + pw_, col] = 1.0 / npx
                col += 1
    return m


@functools.partial(jax.jit, static_argnums=())
def kernel(x, Wq, Wk, Wv, Wo, Wquer, bquer, Wf, Wdw, alpha, beta):
    del alpha, beta
    t = (x.reshape(C, NH, PH, NW, PW)
          .transpose(1, 3, 2, 4, 0)
          .reshape(NPATCH * NPP, C))

    spa_pm, sums = pl.pallas_call(
        _attn_kernel,
        grid=(GRID1,),
        in_specs=[
            pl.BlockSpec((TOK_PER_STEP, C), lambda i: (i, 0)),
            pl.BlockSpec((C, INNER), lambda i: (0, 0)),
            pl.BlockSpec((C, INNER), lambda i: (0, 0)),
            pl.BlockSpec((C, INNER), lambda i: (0, 0)),
            pl.BlockSpec((INNER, C), lambda i: (0, 0)),
        ],
        out_specs=[
            pl.BlockSpec((TOK_PER_STEP, C), lambda i: (i, 0)),
            pl.BlockSpec((PATCHES_PER_STEP, C), lambda i: (i, 0)),
        ],
        out_shape=[
            jax.ShapeDtypeStruct((NPATCH * NPP, C), _F32),
            jax.ShapeDtypeStruct((NPATCH, C), _F32),
        ],
    )(t, Wq, Wk, Wv, Wo)

    pool_m = jnp.asarray(_pool_matrix())
    fc = pl.pallas_call(
        _ctx_kernel,
        out_shape=jax.ShapeDtypeStruct((C, C), _F32),
    )(sums, pool_m, Wf)

    spa = (spa_pm.reshape(NH, NW, PH, PW, C)
           .transpose(4, 0, 2, 1, 3)
           .reshape(C, HW))

    x2d = x.reshape(C, HW)
    out = pl.pallas_call(
        _fmam_kernel,
        grid=(GRID3,),
        in_specs=[
            pl.BlockSpec((C, PIX_PER_STEP), lambda i: (0, i)),
            pl.BlockSpec((C, PIX_PER_STEP), lambda i: (0, i)),
            pl.BlockSpec((C, C), lambda i: (0, 0)),
            pl.BlockSpec((C, C), lambda i: (0, 0)),
            pl.BlockSpec((C, 1), lambda i: (0, 0)),
            pl.BlockSpec((C, 2), lambda i: (0, 0)),
        ],
        out_specs=pl.BlockSpec((C, PIX_PER_STEP), lambda i: (0, i)),
        out_shape=jax.ShapeDtypeStruct((C, HW), _F32),
    )(x2d, spa, fc, Wquer, bquer.reshape(C, 1), Wdw)

    return out.reshape(1, C, 224, 224)
